# baseline (speedup 1.0000x reference)
import functools

import jax
import jax.numpy as jnp
from jax import lax
from jax.experimental import pallas as pl
from jax.experimental.pallas import tpu as pltpu
from jax.experimental.pallas import tpu_sc as plsc

N_NODES = 10000
N_ROWS_PAD = 10240
D = 128
N_EDGES = 320000

NC, NS, L = 2, 16, 16
NW = NC * NS
CHUNK = 128
HCH = CHUNK // 2
EPT = 10240
E_PAD = EPT * NW
N_CHUNKS = EPT // CHUNK
ROWS_PT = N_ROWS_PAD // NS

G = 16
N_GROUPS = N_CHUNKS // G

_mesh = plsc.VectorSubcoreMesh(
    core_axis_name="c", subcore_axis_name="s", num_cores=NC, num_subcores=NS
)


def _scale_chunk(rows, w_v, a):
    def scale_body(g, inner):
        wv = w_v[a, pl.ds(g * L, L)]
        for m in range(L):
            wm = wv[m]
            e = g * L + m
            for j in range(D // L):
                sl = pl.ds(j * L, L)
                rows[e, sl] = rows[e, sl] * wm
        return inner

    lax.fori_loop(0, CHUNK // L, scale_body, 0)


def _gather_chunk(sup_hbm, sidx_v, a, rows, sem):
    pltpu.async_copy(sup_hbm.at[sidx_v.at[a, pl.ds(0, HCH)]],
                     rows.at[pl.ds(0, HCH)], sem)
    pltpu.async_copy(sup_hbm.at[sidx_v.at[a, pl.ds(HCH, HCH)]],
                     rows.at[pl.ds(HCH, HCH)], sem)


def _wait_gather_chunk(sup_hbm, sidx_v, a, rows, sem):
    pltpu.make_async_copy(sup_hbm.at[sidx_v.at[a, pl.ds(0, HCH)]],
                          rows.at[pl.ds(0, HCH)], sem).wait()
    pltpu.make_async_copy(sup_hbm.at[sidx_v.at[a, pl.ds(HCH, HCH)]],
                          rows.at[pl.ds(HCH, HCH)], sem).wait()


@functools.partial(
    pl.kernel,
    out_type=jax.ShapeDtypeStruct((NC, N_ROWS_PAD, D), jnp.float32),
    mesh=_mesh,
    scratch_types=[
        pltpu.VMEM_SHARED((N_ROWS_PAD, D), jnp.float32),
        pltpu.VMEM((G, CHUNK), jnp.int32),
        pltpu.VMEM((G, CHUNK), jnp.int32),
        pltpu.VMEM((G, CHUNK), jnp.float32),
        pltpu.VMEM((CHUNK, D), jnp.float32),
        pltpu.VMEM((CHUNK, D), jnp.float32),
        pltpu.SemaphoreType.DMA,
        pltpu.SemaphoreType.DMA,
        pltpu.SemaphoreType.DMA,
        pltpu.SemaphoreType.DMA,
    ],
)
def _sc_edge_aggregate(sup_hbm, src_hbm, dst_hbm, w_hbm, out_hbm,
                       acc, sidx_v, didx_v, w_v, rows_a, rows_b,
                       sem_ga, sem_gb, sem_sa, sem_sb):
    c = lax.axis_index("c")
    s = lax.axis_index("s")
    wid = s * NC + c
    row0 = s * ROWS_PT

    def zero_body(i, carry):
        for j in range(D // L):
            rows_a[i, pl.ds(j * L, L)] = jnp.zeros((L,), jnp.float32)
        return carry

    lax.fori_loop(0, CHUNK, zero_body, 0)
    for t in range(ROWS_PT // CHUNK):
        pltpu.sync_copy(rows_a, acc.at[pl.ds(row0 + t * CHUNK, CHUNK)])
    plsc.subcore_barrier()

    crow0 = wid * N_CHUNKS
    for grp in range(N_GROUPS):
        g0 = crow0 + grp * G
        pltpu.sync_copy(src_hbm.at[pl.ds(g0, G)], sidx_v)
        pltpu.sync_copy(dst_hbm.at[pl.ds(g0, G)], didx_v)
        pltpu.sync_copy(w_hbm.at[pl.ds(g0, G)], w_v)
        _gather_chunk(sup_hbm, sidx_v, 0, rows_a, sem_ga)
        _gather_chunk(sup_hbm, sidx_v, 1, rows_b, sem_gb)

        def body(t, carry):
            a = 2 * t
            b = a + 1
            _wait_gather_chunk(sup_hbm, sidx_v, a, rows_a, sem_ga)
            _scale_chunk(rows_a, w_v, a)
            sc_a = pltpu.async_copy(rows_a, acc.at[didx_v.at[a]], sem_sa,
                                    add=True)
            _wait_gather_chunk(sup_hbm, sidx_v, b, rows_b, sem_gb)
            _scale_chunk(rows_b, w_v, b)
            sc_b = pltpu.async_copy(rows_b, acc.at[didx_v.at[b]], sem_sb,
                                    add=True)
            sc_a.wait()

            @pl.when(t < G // 2 - 1)
            def _():
                _gather_chunk(sup_hbm, sidx_v, a + 2, rows_a, sem_ga)

            sc_b.wait()

            @pl.when(t < G // 2 - 1)
            def _():
                _gather_chunk(sup_hbm, sidx_v, b + 2, rows_b, sem_gb)

            return carry

        lax.fori_loop(0, G // 2, body, 0)
    plsc.subcore_barrier()

    pltpu.sync_copy(acc.at[pl.ds(row0, ROWS_PT)],
                    out_hbm.at[c, pl.ds(row0, ROWS_PT)])


_BM = 1000


def _tc_matmul_body(x_ref, w_ref, o_ref):
    o_ref[...] = jnp.dot(x_ref[...], w_ref[...],
                         preferred_element_type=jnp.float32)


def _matmul(x, W):
    return pl.pallas_call(
        _tc_matmul_body,
        grid=(N_NODES // _BM,),
        in_specs=[
            pl.BlockSpec((_BM, D), lambda i: (i, 0)),
            pl.BlockSpec((D, D), lambda i: (0, 0)),
        ],
        out_specs=pl.BlockSpec((_BM, D), lambda i: (i, 0)),
        out_shape=jax.ShapeDtypeStruct((N_NODES, D), jnp.float32),
    )(x, W)


def _tc_combine_relu_matmul_body(p_ref, b_ref, w_ref, o_ref):
    x = p_ref[0] + p_ref[1] + b_ref[...]
    o_ref[...] = jnp.dot(jnp.maximum(x, 0.0), w_ref[...],
                         preferred_element_type=jnp.float32)


def _combine_relu_matmul(p, b, W):
    return pl.pallas_call(
        _tc_combine_relu_matmul_body,
        grid=(N_NODES // _BM,),
        in_specs=[
            pl.BlockSpec((NC, _BM, D), lambda i: (0, i, 0)),
            pl.BlockSpec((1, D), lambda i: (0, 0)),
            pl.BlockSpec((D, D), lambda i: (0, 0)),
        ],
        out_specs=pl.BlockSpec((_BM, D), lambda i: (i, 0)),
        out_shape=jax.ShapeDtypeStruct((N_NODES, D), jnp.float32),
    )(p, b.reshape(1, D), W)


def _tc_combine_body(p_ref, b_ref, o_ref):
    o_ref[...] = p_ref[0] + p_ref[1] + b_ref[...]


def _combine(p, b):
    return pl.pallas_call(
        _tc_combine_body,
        grid=(N_NODES // _BM,),
        in_specs=[
            pl.BlockSpec((NC, _BM, D), lambda i: (0, i, 0)),
            pl.BlockSpec((1, D), lambda i: (0, 0)),
        ],
        out_specs=pl.BlockSpec((_BM, D), lambda i: (i, 0)),
        out_shape=jax.ShapeDtypeStruct((N_NODES, D), jnp.float32),
    )(p, b.reshape(1, D))


def kernel(feat, edge_index, edge_weight, W0, b0, W1, b1):
    src = edge_index[0].astype(jnp.int32)
    dst = edge_index[1].astype(jnp.int32)
    w = edge_weight.astype(jnp.float32)
    pad = E_PAD - N_EDGES
    src = jnp.concatenate([src, jnp.zeros((pad,), jnp.int32)])
    dst = jnp.concatenate([dst, jnp.zeros((pad,), jnp.int32)])
    w = jnp.concatenate([w, jnp.zeros((pad,), jnp.float32)])
    src = src.reshape(E_PAD // CHUNK, CHUNK)
    dst = dst.reshape(E_PAD // CHUNK, CHUNK)
    w = w.reshape(E_PAD // CHUNK, CHUNK)

    sup0 = _matmul(feat, W0)
    p0 = _sc_edge_aggregate(sup0, src, dst, w)
    sup1 = _combine_relu_matmul(p0[:, :N_NODES], b0, W1)
    p1 = _sc_edge_aggregate(sup1, src, dst, w)
    return _combine(p1[:, :N_NODES], b1)


# speedup vs baseline: 1.0160x; 1.0160x over previous
import functools

import jax
import jax.numpy as jnp
from jax import lax
from jax.experimental import pallas as pl
from jax.experimental.pallas import tpu as pltpu
from jax.experimental.pallas import tpu_sc as plsc

N_NODES = 10000
N_ROWS_PAD = 10240
D = 128
N_EDGES = 320000

NC, NS, L = 2, 16, 16
NW = NC * NS
CHUNK = 128
HCH = CHUNK // 2
EPT = 10240
E_PAD = EPT * NW
N_CHUNKS = EPT // CHUNK
ROWS_PT = N_ROWS_PAD // NS

G = 16
N_GROUPS = N_CHUNKS // G

_mesh = plsc.VectorSubcoreMesh(
    core_axis_name="c", subcore_axis_name="s", num_cores=NC, num_subcores=NS
)


def _scale_chunk(rows, w_v, a):
    def scale_body(g, inner):
        wv = w_v[a, pl.ds(g * L, L)]
        for m in range(L):
            wm = wv[m]
            e = g * L + m
            for j in range(D // L):
                sl = pl.ds(j * L, L)
                rows[e, sl] = rows[e, sl] * wm
        return inner

    lax.fori_loop(0, CHUNK // L, scale_body, 0)


def _gather_chunk(sup_c, sidx_v, a, rows, sem):
    pltpu.async_copy(sup_c.at[sidx_v.at[a, pl.ds(0, HCH)]],
                     rows.at[pl.ds(0, HCH)], sem)
    pltpu.async_copy(sup_c.at[sidx_v.at[a, pl.ds(HCH, HCH)]],
                     rows.at[pl.ds(HCH, HCH)], sem)


def _wait_gather_chunk(sup_c, sidx_v, a, rows, sem):
    pltpu.make_async_copy(sup_c.at[sidx_v.at[a, pl.ds(0, HCH)]],
                          rows.at[pl.ds(0, HCH)], sem).wait()
    pltpu.make_async_copy(sup_c.at[sidx_v.at[a, pl.ds(HCH, HCH)]],
                          rows.at[pl.ds(HCH, HCH)], sem).wait()


@functools.partial(
    pl.kernel,
    out_type=jax.ShapeDtypeStruct((NC, N_ROWS_PAD, D), jnp.float32),
    mesh=_mesh,
    scratch_types=[
        pltpu.VMEM_SHARED((N_ROWS_PAD, D), jnp.float32),
        pltpu.VMEM((G, CHUNK), jnp.int32),
        pltpu.VMEM((G, CHUNK), jnp.int32),
        pltpu.VMEM((G, CHUNK), jnp.float32),
        pltpu.VMEM((CHUNK, D), jnp.float32),
        pltpu.VMEM((CHUNK, D), jnp.float32),
        pltpu.SemaphoreType.DMA,
        pltpu.SemaphoreType.DMA,
        pltpu.SemaphoreType.DMA,
        pltpu.SemaphoreType.DMA,
    ],
)
def _sc_edge_aggregate(sup_hbm, src_hbm, dst_hbm, w_hbm, out_hbm,
                       acc, sidx_v, didx_v, w_v, rows_a, rows_b,
                       sem_ga, sem_gb, sem_sa, sem_sb):
    c = lax.axis_index("c")
    s = lax.axis_index("s")
    wid = s * NC + c
    row0 = s * ROWS_PT
    sup_c = sup_hbm.at[c]

    def zero_body(i, carry):
        for j in range(D // L):
            rows_a[i, pl.ds(j * L, L)] = jnp.zeros((L,), jnp.float32)
        return carry

    lax.fori_loop(0, CHUNK, zero_body, 0)
    for t in range(ROWS_PT // CHUNK):
        pltpu.sync_copy(rows_a, acc.at[pl.ds(row0 + t * CHUNK, CHUNK)])
    plsc.subcore_barrier()

    crow0 = wid * N_CHUNKS
    for grp in range(N_GROUPS):
        g0 = crow0 + grp * G
        pltpu.sync_copy(src_hbm.at[pl.ds(g0, G)], sidx_v)
        pltpu.sync_copy(dst_hbm.at[pl.ds(g0, G)], didx_v)
        pltpu.sync_copy(w_hbm.at[pl.ds(g0, G)], w_v)
        _gather_chunk(sup_c, sidx_v, 0, rows_a, sem_ga)
        _gather_chunk(sup_c, sidx_v, 1, rows_b, sem_gb)

        def body(t, carry):
            a = 2 * t
            b = a + 1
            _wait_gather_chunk(sup_c, sidx_v, a, rows_a, sem_ga)
            _scale_chunk(rows_a, w_v, a)
            sc_a = pltpu.async_copy(rows_a, acc.at[didx_v.at[a]], sem_sa,
                                    add=True)
            _wait_gather_chunk(sup_c, sidx_v, b, rows_b, sem_gb)
            _scale_chunk(rows_b, w_v, b)
            sc_b = pltpu.async_copy(rows_b, acc.at[didx_v.at[b]], sem_sb,
                                    add=True)
            sc_a.wait()

            @pl.when(t < G // 2 - 1)
            def _():
                _gather_chunk(sup_c, sidx_v, a + 2, rows_a, sem_ga)

            sc_b.wait()

            @pl.when(t < G // 2 - 1)
            def _():
                _gather_chunk(sup_c, sidx_v, b + 2, rows_b, sem_gb)

            return carry

        lax.fori_loop(0, G // 2, body, 0)
    plsc.subcore_barrier()

    pltpu.sync_copy(acc.at[pl.ds(row0, ROWS_PT)],
                    out_hbm.at[c, pl.ds(row0, ROWS_PT)])


_BM = 1000


def _tc_matmul_body(x_ref, w_ref, o_ref):
    y = jnp.dot(x_ref[...], w_ref[...], preferred_element_type=jnp.float32)
    o_ref[0] = y
    o_ref[1] = y


def _matmul(x, W):
    return pl.pallas_call(
        _tc_matmul_body,
        grid=(N_NODES // _BM,),
        in_specs=[
            pl.BlockSpec((_BM, D), lambda i: (i, 0)),
            pl.BlockSpec((D, D), lambda i: (0, 0)),
        ],
        out_specs=pl.BlockSpec((NC, _BM, D), lambda i: (0, i, 0)),
        out_shape=jax.ShapeDtypeStruct((NC, N_NODES, D), jnp.float32),
    )(x, W)


def _tc_combine_relu_matmul_body(p_ref, b_ref, w_ref, o_ref):
    x = p_ref[0] + p_ref[1] + b_ref[...]
    y = jnp.dot(jnp.maximum(x, 0.0), w_ref[...],
                preferred_element_type=jnp.float32)
    o_ref[0] = y
    o_ref[1] = y


def _combine_relu_matmul(p, b, W):
    return pl.pallas_call(
        _tc_combine_relu_matmul_body,
        grid=(N_NODES // _BM,),
        in_specs=[
            pl.BlockSpec((NC, _BM, D), lambda i: (0, i, 0)),
            pl.BlockSpec((1, D), lambda i: (0, 0)),
            pl.BlockSpec((D, D), lambda i: (0, 0)),
        ],
        out_specs=pl.BlockSpec((NC, _BM, D), lambda i: (0, i, 0)),
        out_shape=jax.ShapeDtypeStruct((NC, N_NODES, D), jnp.float32),
    )(p, b.reshape(1, D), W)


def _tc_combine_body(p_ref, b_ref, o_ref):
    o_ref[...] = p_ref[0] + p_ref[1] + b_ref[...]


def _combine(p, b):
    return pl.pallas_call(
        _tc_combine_body,
        grid=(N_NODES // _BM,),
        in_specs=[
            pl.BlockSpec((NC, _BM, D), lambda i: (0, i, 0)),
            pl.BlockSpec((1, D), lambda i: (0, 0)),
        ],
        out_specs=pl.BlockSpec((_BM, D), lambda i: (i, 0)),
        out_shape=jax.ShapeDtypeStruct((N_NODES, D), jnp.float32),
    )(p, b.reshape(1, D))


def kernel(feat, edge_index, edge_weight, W0, b0, W1, b1):
    src = edge_index[0].astype(jnp.int32)
    dst = edge_index[1].astype(jnp.int32)
    w = edge_weight.astype(jnp.float32)
    pad = E_PAD - N_EDGES
    src = jnp.concatenate([src, jnp.zeros((pad,), jnp.int32)])
    dst = jnp.concatenate([dst, jnp.zeros((pad,), jnp.int32)])
    w = jnp.concatenate([w, jnp.zeros((pad,), jnp.float32)])
    src = src.reshape(E_PAD // CHUNK, CHUNK)
    dst = dst.reshape(E_PAD // CHUNK, CHUNK)
    w = w.reshape(E_PAD // CHUNK, CHUNK)

    sup0 = _matmul(feat, W0)
    p0 = _sc_edge_aggregate(sup0, src, dst, w)
    sup1 = _combine_relu_matmul(p0[:, :N_NODES], b0, W1)
    p1 = _sc_edge_aggregate(sup1, src, dst, w)
    return _combine(p1[:, :N_NODES], b1)


# speedup vs baseline: 1.7371x; 1.7098x over previous
import functools

import jax
import jax.numpy as jnp
from jax import lax
from jax.experimental import pallas as pl
from jax.experimental.pallas import tpu as pltpu
from jax.experimental.pallas import tpu_sc as plsc

N_NODES = 10000
N_ROWS_PAD = 10240
D = 128
N_EDGES = 320000

NC, NS, L = 2, 16, 16
NW = NC * NS
CHUNK = 128
EPT = 10240
E_PAD = EPT * NW
N_CHUNKS = EPT // CHUNK
ROWS_PT = N_ROWS_PAD // NS

G = 16
N_GROUPS = N_CHUNKS // G

_mesh = plsc.VectorSubcoreMesh(
    core_axis_name="c", subcore_axis_name="s", num_cores=NC, num_subcores=NS
)


def _scale_chunk(rows, w_v, a):
    def scale_body(g, inner):
        wv = w_v[a, pl.ds(g * L, L)]
        for m in range(L):
            wm = wv[m]
            e = g * L + m
            for j in range(D // L):
                sl = pl.ds(j * L, L)
                rows[e, sl] = rows[e, sl] * wm
        return inner

    lax.fori_loop(0, CHUNK // L, scale_body, 0)


@functools.partial(
    pl.kernel,
    out_type=jax.ShapeDtypeStruct((E_PAD, D), jnp.float32),
    mesh=_mesh,
    scratch_types=[
        pltpu.VMEM_SHARED((N_ROWS_PAD, D), jnp.float32),
        pltpu.VMEM((G, CHUNK), jnp.int32),
        pltpu.VMEM((G, CHUNK), jnp.float32),
        pltpu.VMEM((CHUNK, D), jnp.float32),
        pltpu.VMEM((CHUNK, D), jnp.float32),
        pltpu.SemaphoreType.DMA,
        pltpu.SemaphoreType.DMA,
        pltpu.SemaphoreType.DMA,
        pltpu.SemaphoreType.DMA,
    ],
)
def _sc_gather_scale(sup_hbm, src_hbm, w_hbm, msgs_hbm,
                     sup_s, sidx_v, w_v, rows_a, rows_b,
                     sem_ga, sem_gb, sem_wa, sem_wb):
    c = lax.axis_index("c")
    s = lax.axis_index("s")
    wid = s * NC + c
    row0 = s * ROWS_PT

    @pl.when(s < NS - 1)
    def _():
        for t in range(ROWS_PT // CHUNK):
            pltpu.sync_copy(sup_hbm.at[pl.ds(row0 + t * CHUNK, CHUNK)], rows_a)
            pltpu.sync_copy(rows_a, sup_s.at[pl.ds(row0 + t * CHUNK, CHUNK)])

    @pl.when(s == NS - 1)
    def _():
        for t in range(3):
            pltpu.sync_copy(sup_hbm.at[pl.ds(row0 + t * CHUNK, CHUNK)], rows_a)
            pltpu.sync_copy(rows_a, sup_s.at[pl.ds(row0 + t * CHUNK, CHUNK)])
        pltpu.sync_copy(sup_hbm.at[pl.ds(row0 + 3 * CHUNK, 16)],
                        rows_a.at[pl.ds(0, 16)])
        pltpu.sync_copy(rows_a.at[pl.ds(0, 16)],
                        sup_s.at[pl.ds(row0 + 3 * CHUNK, 16)])

    plsc.subcore_barrier()

    crow0 = wid * N_CHUNKS
    for grp in range(N_GROUPS):
        g0 = crow0 + grp * G
        pltpu.sync_copy(src_hbm.at[pl.ds(g0, G)], sidx_v)
        pltpu.sync_copy(w_hbm.at[pl.ds(g0, G)], w_v)
        pltpu.async_copy(sup_s.at[sidx_v.at[0]], rows_a, sem_ga)
        pltpu.async_copy(sup_s.at[sidx_v.at[1]], rows_b, sem_gb)

        def body(t, carry):
            a = 2 * t
            b = a + 1
            off = (g0 + a) * CHUNK
            pltpu.make_async_copy(sup_s.at[sidx_v.at[a]], rows_a,
                                  sem_ga).wait()
            _scale_chunk(rows_a, w_v, a)
            wr_a = pltpu.async_copy(rows_a, msgs_hbm.at[pl.ds(off, CHUNK)],
                                    sem_wa)
            pltpu.make_async_copy(sup_s.at[sidx_v.at[b]], rows_b,
                                  sem_gb).wait()
            _scale_chunk(rows_b, w_v, b)
            wr_b = pltpu.async_copy(rows_b,
                                    msgs_hbm.at[pl.ds(off + CHUNK, CHUNK)],
                                    sem_wb)
            wr_a.wait()

            @pl.when(t < G // 2 - 1)
            def _():
                pltpu.async_copy(sup_s.at[sidx_v.at[a + 2]], rows_a, sem_ga)

            wr_b.wait()

            @pl.when(t < G // 2 - 1)
            def _():
                pltpu.async_copy(sup_s.at[sidx_v.at[b + 2]], rows_b, sem_gb)

            return carry

        lax.fori_loop(0, G // 2, body, 0)


@functools.partial(
    pl.kernel,
    out_type=jax.ShapeDtypeStruct((NC, N_ROWS_PAD, D), jnp.float32),
    mesh=_mesh,
    scratch_types=[
        pltpu.VMEM_SHARED((N_ROWS_PAD, D), jnp.float32),
        pltpu.VMEM((G, CHUNK), jnp.int32),
        pltpu.VMEM((CHUNK, D), jnp.float32),
        pltpu.VMEM((CHUNK, D), jnp.float32),
        pltpu.SemaphoreType.DMA,
        pltpu.SemaphoreType.DMA,
        pltpu.SemaphoreType.DMA,
        pltpu.SemaphoreType.DMA,
    ],
)
def _sc_scatter(msgs_hbm, dst_hbm, out_hbm,
                acc, didx_v, rows_a, rows_b,
                sem_ra, sem_rb, sem_sa, sem_sb):
    c = lax.axis_index("c")
    s = lax.axis_index("s")
    wid = s * NC + c
    row0 = s * ROWS_PT

    def zero_body(i, carry):
        for j in range(D // L):
            rows_a[i, pl.ds(j * L, L)] = jnp.zeros((L,), jnp.float32)
        return carry

    lax.fori_loop(0, CHUNK, zero_body, 0)
    for t in range(ROWS_PT // CHUNK):
        pltpu.sync_copy(rows_a, acc.at[pl.ds(row0 + t * CHUNK, CHUNK)])
    plsc.subcore_barrier()

    crow0 = wid * N_CHUNKS
    for grp in range(N_GROUPS):
        g0 = crow0 + grp * G
        pltpu.sync_copy(dst_hbm.at[pl.ds(g0, G)], didx_v)
        pltpu.async_copy(msgs_hbm.at[pl.ds(g0 * CHUNK, CHUNK)], rows_a,
                         sem_ra)
        pltpu.async_copy(msgs_hbm.at[pl.ds((g0 + 1) * CHUNK, CHUNK)], rows_b,
                         sem_rb)

        def body(t, carry):
            a = 2 * t
            b = a + 1
            off = (g0 + a) * CHUNK
            pltpu.make_async_copy(msgs_hbm.at[pl.ds(off, CHUNK)], rows_a,
                                  sem_ra).wait()
            sc_a = pltpu.async_copy(rows_a, acc.at[didx_v.at[a]], sem_sa,
                                    add=True)
            pltpu.make_async_copy(msgs_hbm.at[pl.ds(off + CHUNK, CHUNK)],
                                  rows_b, sem_rb).wait()
            sc_b = pltpu.async_copy(rows_b, acc.at[didx_v.at[b]], sem_sb,
                                    add=True)
            sc_a.wait()

            @pl.when(t < G // 2 - 1)
            def _():
                pltpu.async_copy(msgs_hbm.at[pl.ds(off + 2 * CHUNK, CHUNK)],
                                 rows_a, sem_ra)

            sc_b.wait()

            @pl.when(t < G // 2 - 1)
            def _():
                pltpu.async_copy(msgs_hbm.at[pl.ds(off + 3 * CHUNK, CHUNK)],
                                 rows_b, sem_rb)

            return carry

        lax.fori_loop(0, G // 2, body, 0)
    plsc.subcore_barrier()

    pltpu.sync_copy(acc.at[pl.ds(row0, ROWS_PT)],
                    out_hbm.at[c, pl.ds(row0, ROWS_PT)])


_BM = 1000


def _tc_matmul_body(x_ref, w_ref, o_ref):
    o_ref[...] = jnp.dot(x_ref[...], w_ref[...],
                         preferred_element_type=jnp.float32)


def _matmul(x, W):
    return pl.pallas_call(
        _tc_matmul_body,
        grid=(N_NODES // _BM,),
        in_specs=[
            pl.BlockSpec((_BM, D), lambda i: (i, 0)),
            pl.BlockSpec((D, D), lambda i: (0, 0)),
        ],
        out_specs=pl.BlockSpec((_BM, D), lambda i: (i, 0)),
        out_shape=jax.ShapeDtypeStruct((N_NODES, D), jnp.float32),
    )(x, W)


def _tc_combine_relu_matmul_body(p_ref, b_ref, w_ref, o_ref):
    x = p_ref[0] + p_ref[1] + b_ref[...]
    o_ref[...] = jnp.dot(jnp.maximum(x, 0.0), w_ref[...],
                         preferred_element_type=jnp.float32)


def _combine_relu_matmul(p, b, W):
    return pl.pallas_call(
        _tc_combine_relu_matmul_body,
        grid=(N_NODES // _BM,),
        in_specs=[
            pl.BlockSpec((NC, _BM, D), lambda i: (0, i, 0)),
            pl.BlockSpec((1, D), lambda i: (0, 0)),
            pl.BlockSpec((D, D), lambda i: (0, 0)),
        ],
        out_specs=pl.BlockSpec((_BM, D), lambda i: (i, 0)),
        out_shape=jax.ShapeDtypeStruct((N_NODES, D), jnp.float32),
    )(p, b.reshape(1, D), W)


def _tc_combine_body(p_ref, b_ref, o_ref):
    o_ref[...] = p_ref[0] + p_ref[1] + b_ref[...]


def _combine(p, b):
    return pl.pallas_call(
        _tc_combine_body,
        grid=(N_NODES // _BM,),
        in_specs=[
            pl.BlockSpec((NC, _BM, D), lambda i: (0, i, 0)),
            pl.BlockSpec((1, D), lambda i: (0, 0)),
        ],
        out_specs=pl.BlockSpec((_BM, D), lambda i: (i, 0)),
        out_shape=jax.ShapeDtypeStruct((N_NODES, D), jnp.float32),
    )(p, b.reshape(1, D))


def kernel(feat, edge_index, edge_weight, W0, b0, W1, b1):
    src = edge_index[0].astype(jnp.int32)
    dst = edge_index[1].astype(jnp.int32)
    w = edge_weight.astype(jnp.float32)
    pad = E_PAD - N_EDGES
    src = jnp.concatenate([src, jnp.zeros((pad,), jnp.int32)])
    dst = jnp.concatenate([dst, jnp.zeros((pad,), jnp.int32)])
    w = jnp.concatenate([w, jnp.zeros((pad,), jnp.float32)])
    src = src.reshape(E_PAD // CHUNK, CHUNK)
    dst = dst.reshape(E_PAD // CHUNK, CHUNK)
    w = w.reshape(E_PAD // CHUNK, CHUNK)

    sup0 = _matmul(feat, W0)
    m0 = _sc_gather_scale(sup0, src, w)
    p0 = _sc_scatter(m0, dst)
    sup1 = _combine_relu_matmul(p0[:, :N_NODES], b0, W1)
    m1 = _sc_gather_scale(sup1, src, w)
    p1 = _sc_scatter(m1, dst)
    return _combine(p1[:, :N_NODES], b1)


# speedup vs baseline: 1.8109x; 1.0425x over previous
import functools

import jax
import jax.numpy as jnp
from jax import lax
from jax.experimental import pallas as pl
from jax.experimental.pallas import tpu as pltpu
from jax.experimental.pallas import tpu_sc as plsc

N_NODES = 10000
N_ROWS_PAD = 10240
D = 128
N_EDGES = 320000

NC, NS, L = 2, 16, 16
NW = NC * NS
CHUNK = 128
EPT = 10240
E_PAD = EPT * NW
N_CHUNKS = EPT // CHUNK
ROWS_PT = N_ROWS_PAD // NS

G = 40
N_GROUPS = N_CHUNKS // G

_mesh = plsc.VectorSubcoreMesh(
    core_axis_name="c", subcore_axis_name="s", num_cores=NC, num_subcores=NS
)


def _scale_chunk(rows, w_v, a):
    def scale_body(g, inner):
        wv = w_v[a, pl.ds(g * L, L)]
        for m in range(L):
            wm = wv[m]
            e = g * L + m
            for j in range(D // L):
                sl = pl.ds(j * L, L)
                rows[e, sl] = rows[e, sl] * wm
        return inner

    lax.fori_loop(0, CHUNK // L, scale_body, 0)


@functools.partial(
    pl.kernel,
    out_type=jax.ShapeDtypeStruct((E_PAD, D), jnp.float32),
    mesh=_mesh,
    scratch_types=[
        pltpu.VMEM_SHARED((N_ROWS_PAD, D), jnp.float32),
        pltpu.VMEM((G, CHUNK), jnp.int32),
        pltpu.VMEM((G, CHUNK), jnp.float32),
        pltpu.VMEM((CHUNK, D), jnp.float32),
        pltpu.VMEM((CHUNK, D), jnp.float32),
        pltpu.SemaphoreType.DMA,
        pltpu.SemaphoreType.DMA,
        pltpu.SemaphoreType.DMA,
        pltpu.SemaphoreType.DMA,
    ],
)
def _sc_gather_scale(sup_hbm, src_hbm, w_hbm, msgs_hbm,
                     sup_s, sidx_v, w_v, rows_a, rows_b,
                     sem_ga, sem_gb, sem_wa, sem_wb):
    c = lax.axis_index("c")
    s = lax.axis_index("s")
    wid = s * NC + c
    row0 = s * ROWS_PT

    @pl.when(s < NS - 1)
    def _():
        for t in range(ROWS_PT // CHUNK):
            pltpu.sync_copy(sup_hbm.at[pl.ds(row0 + t * CHUNK, CHUNK)], rows_a)
            pltpu.sync_copy(rows_a, sup_s.at[pl.ds(row0 + t * CHUNK, CHUNK)])

    @pl.when(s == NS - 1)
    def _():
        for t in range(3):
            pltpu.sync_copy(sup_hbm.at[pl.ds(row0 + t * CHUNK, CHUNK)], rows_a)
            pltpu.sync_copy(rows_a, sup_s.at[pl.ds(row0 + t * CHUNK, CHUNK)])
        pltpu.sync_copy(sup_hbm.at[pl.ds(row0 + 3 * CHUNK, 16)],
                        rows_a.at[pl.ds(0, 16)])
        pltpu.sync_copy(rows_a.at[pl.ds(0, 16)],
                        sup_s.at[pl.ds(row0 + 3 * CHUNK, 16)])

    plsc.subcore_barrier()

    crow0 = wid * N_CHUNKS
    for grp in range(N_GROUPS):
        g0 = crow0 + grp * G
        pltpu.sync_copy(src_hbm.at[pl.ds(g0, G)], sidx_v)
        pltpu.sync_copy(w_hbm.at[pl.ds(g0, G)], w_v)
        pltpu.async_copy(sup_s.at[sidx_v.at[0]], rows_a, sem_ga)
        pltpu.async_copy(sup_s.at[sidx_v.at[1]], rows_b, sem_gb)

        def body(t, carry):
            a = 2 * t
            b = a + 1
            off = (g0 + a) * CHUNK
            pltpu.make_async_copy(sup_s.at[sidx_v.at[a]], rows_a,
                                  sem_ga).wait()
            _scale_chunk(rows_a, w_v, a)
            wr_a = pltpu.async_copy(rows_a, msgs_hbm.at[pl.ds(off, CHUNK)],
                                    sem_wa)
            pltpu.make_async_copy(sup_s.at[sidx_v.at[b]], rows_b,
                                  sem_gb).wait()
            _scale_chunk(rows_b, w_v, b)
            wr_b = pltpu.async_copy(rows_b,
                                    msgs_hbm.at[pl.ds(off + CHUNK, CHUNK)],
                                    sem_wb)
            wr_a.wait()

            @pl.when(t < G // 2 - 1)
            def _():
                pltpu.async_copy(sup_s.at[sidx_v.at[a + 2]], rows_a, sem_ga)

            wr_b.wait()

            @pl.when(t < G // 2 - 1)
            def _():
                pltpu.async_copy(sup_s.at[sidx_v.at[b + 2]], rows_b, sem_gb)

            return carry

        lax.fori_loop(0, G // 2, body, 0)


@functools.partial(
    pl.kernel,
    out_type=jax.ShapeDtypeStruct((NC, N_ROWS_PAD, D), jnp.float32),
    mesh=_mesh,
    scratch_types=[
        pltpu.VMEM_SHARED((N_ROWS_PAD, D), jnp.float32),
        pltpu.VMEM((G, CHUNK), jnp.int32),
        pltpu.VMEM((CHUNK, D), jnp.float32),
        pltpu.VMEM((CHUNK, D), jnp.float32),
        pltpu.SemaphoreType.DMA,
        pltpu.SemaphoreType.DMA,
        pltpu.SemaphoreType.DMA,
        pltpu.SemaphoreType.DMA,
    ],
)
def _sc_scatter(msgs_hbm, dst_hbm, out_hbm,
                acc, didx_v, rows_a, rows_b,
                sem_ra, sem_rb, sem_sa, sem_sb):
    c = lax.axis_index("c")
    s = lax.axis_index("s")
    wid = s * NC + c
    row0 = s * ROWS_PT

    def zero_body(i, carry):
        for j in range(D // L):
            rows_a[i, pl.ds(j * L, L)] = jnp.zeros((L,), jnp.float32)
        return carry

    lax.fori_loop(0, CHUNK, zero_body, 0)
    for t in range(ROWS_PT // CHUNK):
        pltpu.sync_copy(rows_a, acc.at[pl.ds(row0 + t * CHUNK, CHUNK)])
    plsc.subcore_barrier()

    crow0 = wid * N_CHUNKS
    for grp in range(N_GROUPS):
        g0 = crow0 + grp * G
        pltpu.sync_copy(dst_hbm.at[pl.ds(g0, G)], didx_v)
        pltpu.async_copy(msgs_hbm.at[pl.ds(g0 * CHUNK, CHUNK)], rows_a,
                         sem_ra)
        pltpu.async_copy(msgs_hbm.at[pl.ds((g0 + 1) * CHUNK, CHUNK)], rows_b,
                         sem_rb)

        def body(t, carry):
            a = 2 * t
            b = a + 1
            off = (g0 + a) * CHUNK
            pltpu.make_async_copy(msgs_hbm.at[pl.ds(off, CHUNK)], rows_a,
                                  sem_ra).wait()
            sc_a = pltpu.async_copy(rows_a, acc.at[didx_v.at[a]], sem_sa,
                                    add=True)
            pltpu.make_async_copy(msgs_hbm.at[pl.ds(off + CHUNK, CHUNK)],
                                  rows_b, sem_rb).wait()
            sc_b = pltpu.async_copy(rows_b, acc.at[didx_v.at[b]], sem_sb,
                                    add=True)
            sc_a.wait()

            @pl.when(t < G // 2 - 1)
            def _():
                pltpu.async_copy(msgs_hbm.at[pl.ds(off + 2 * CHUNK, CHUNK)],
                                 rows_a, sem_ra)

            sc_b.wait()

            @pl.when(t < G // 2 - 1)
            def _():
                pltpu.async_copy(msgs_hbm.at[pl.ds(off + 3 * CHUNK, CHUNK)],
                                 rows_b, sem_rb)

            return carry

        lax.fori_loop(0, G // 2, body, 0)
    plsc.subcore_barrier()

    pltpu.sync_copy(acc.at[pl.ds(row0, ROWS_PT)],
                    out_hbm.at[c, pl.ds(row0, ROWS_PT)])


_BM = 1000


def _tc_matmul_body(x_ref, w_ref, o_ref):
    o_ref[...] = jnp.dot(x_ref[...], w_ref[...],
                         preferred_element_type=jnp.float32)


def _matmul(x, W):
    return pl.pallas_call(
        _tc_matmul_body,
        grid=(N_NODES // _BM,),
        in_specs=[
            pl.BlockSpec((_BM, D), lambda i: (i, 0)),
            pl.BlockSpec((D, D), lambda i: (0, 0)),
        ],
        out_specs=pl.BlockSpec((_BM, D), lambda i: (i, 0)),
        out_shape=jax.ShapeDtypeStruct((N_NODES, D), jnp.float32),
    )(x, W)


def _tc_combine_relu_matmul_body(p_ref, b_ref, w_ref, o_ref):
    x = p_ref[0] + p_ref[1] + b_ref[...]
    o_ref[...] = jnp.dot(jnp.maximum(x, 0.0), w_ref[...],
                         preferred_element_type=jnp.float32)


def _combine_relu_matmul(p, b, W):
    return pl.pallas_call(
        _tc_combine_relu_matmul_body,
        grid=(N_NODES // _BM,),
        in_specs=[
            pl.BlockSpec((NC, _BM, D), lambda i: (0, i, 0)),
            pl.BlockSpec((1, D), lambda i: (0, 0)),
            pl.BlockSpec((D, D), lambda i: (0, 0)),
        ],
        out_specs=pl.BlockSpec((_BM, D), lambda i: (i, 0)),
        out_shape=jax.ShapeDtypeStruct((N_NODES, D), jnp.float32),
    )(p, b.reshape(1, D), W)


def _tc_combine_body(p_ref, b_ref, o_ref):
    o_ref[...] = p_ref[0] + p_ref[1] + b_ref[...]


def _combine(p, b):
    return pl.pallas_call(
        _tc_combine_body,
        grid=(N_NODES // _BM,),
        in_specs=[
            pl.BlockSpec((NC, _BM, D), lambda i: (0, i, 0)),
            pl.BlockSpec((1, D), lambda i: (0, 0)),
        ],
        out_specs=pl.BlockSpec((_BM, D), lambda i: (i, 0)),
        out_shape=jax.ShapeDtypeStruct((N_NODES, D), jnp.float32),
    )(p, b.reshape(1, D))


def kernel(feat, edge_index, edge_weight, W0, b0, W1, b1):
    src = edge_index[0].astype(jnp.int32)
    dst = edge_index[1].astype(jnp.int32)
    w = edge_weight.astype(jnp.float32)
    pad = E_PAD - N_EDGES
    src = jnp.concatenate([src, jnp.zeros((pad,), jnp.int32)])
    dst = jnp.concatenate([dst, jnp.zeros((pad,), jnp.int32)])
    w = jnp.concatenate([w, jnp.zeros((pad,), jnp.float32)])
    src = src.reshape(E_PAD // CHUNK, CHUNK)
    dst = dst.reshape(E_PAD // CHUNK, CHUNK)
    w = w.reshape(E_PAD // CHUNK, CHUNK)

    sup0 = _matmul(feat, W0)
    m0 = _sc_gather_scale(sup0, src, w)
    p0 = _sc_scatter(m0, dst)
    sup1 = _combine_relu_matmul(p0[:, :N_NODES], b0, W1)
    m1 = _sc_gather_scale(sup1, src, w)
    p1 = _sc_scatter(m1, dst)
    return _combine(p1[:, :N_NODES], b1)


# speedup vs baseline: 2.0503x; 1.1322x over previous
import functools

import jax
import jax.numpy as jnp
from jax import lax
from jax.experimental import pallas as pl
from jax.experimental.pallas import tpu as pltpu
from jax.experimental.pallas import tpu_sc as plsc

N_NODES = 10000
N_ROWS_PAD = 10240
D = 128
N_EDGES = 320000

NC, NS, L = 2, 16, 16
NW = NC * NS
CHUNK = 128
EPT = 10240
E_PAD = EPT * NW
N_CHUNKS = EPT // CHUNK
ROWS_PT = N_ROWS_PAD // NS

G = 40
N_GROUPS = N_CHUNKS // G

_mesh = plsc.VectorSubcoreMesh(
    core_axis_name="c", subcore_axis_name="s", num_cores=NC, num_subcores=NS
)


def _scale_chunk(rows, w_v, a):
    def scale_body(g, inner):
        wv = w_v[a, pl.ds(g * L, L)]
        for m in range(L):
            wm = wv[m]
            e = g * L + m
            for j in range(D // L):
                sl = pl.ds(j * L, L)
                rows[e, sl] = rows[e, sl] * wm
        return inner

    lax.fori_loop(0, CHUNK // L, scale_body, 0)


@functools.partial(
    pl.kernel,
    out_type=jax.ShapeDtypeStruct((E_PAD, D), jnp.float32),
    mesh=_mesh,
    scratch_types=[
        pltpu.VMEM_SHARED((N_ROWS_PAD, D), jnp.float32),
        pltpu.VMEM((G, CHUNK), jnp.int32),
        pltpu.VMEM((G, CHUNK), jnp.float32),
        pltpu.VMEM((CHUNK, D), jnp.float32),
        pltpu.VMEM((CHUNK, D), jnp.float32),
        pltpu.SemaphoreType.DMA,
        pltpu.SemaphoreType.DMA,
        pltpu.SemaphoreType.DMA,
        pltpu.SemaphoreType.DMA,
    ],
)
def _sc_gather_scale(sup_hbm, src_hbm, w_hbm, msgs_hbm,
                     sup_s, sidx_v, w_v, rows_a, rows_b,
                     sem_ga, sem_gb, sem_wa, sem_wb):
    c = lax.axis_index("c")
    s = lax.axis_index("s")
    wid = s * NC + c
    row0 = s * ROWS_PT

    @pl.when(s < NS - 1)
    def _():
        for t in range(ROWS_PT // CHUNK):
            pltpu.sync_copy(sup_hbm.at[pl.ds(row0 + t * CHUNK, CHUNK)], rows_a)
            pltpu.sync_copy(rows_a, sup_s.at[pl.ds(row0 + t * CHUNK, CHUNK)])

    @pl.when(s == NS - 1)
    def _():
        for t in range(3):
            pltpu.sync_copy(sup_hbm.at[pl.ds(row0 + t * CHUNK, CHUNK)], rows_a)
            pltpu.sync_copy(rows_a, sup_s.at[pl.ds(row0 + t * CHUNK, CHUNK)])
        pltpu.sync_copy(sup_hbm.at[pl.ds(row0 + 3 * CHUNK, 16)],
                        rows_a.at[pl.ds(0, 16)])
        pltpu.sync_copy(rows_a.at[pl.ds(0, 16)],
                        sup_s.at[pl.ds(row0 + 3 * CHUNK, 16)])

    plsc.subcore_barrier()

    crow0 = wid * N_CHUNKS
    for grp in range(N_GROUPS):
        g0 = crow0 + grp * G
        pltpu.sync_copy(src_hbm.at[pl.ds(g0, G)], sidx_v)
        pltpu.sync_copy(w_hbm.at[pl.ds(g0, G)], w_v)
        pltpu.async_copy(sup_s.at[sidx_v.at[0]], rows_a, sem_ga)
        pltpu.async_copy(sup_s.at[sidx_v.at[1]], rows_b, sem_gb)

        def body(t, carry):
            a = 2 * t
            b = a + 1
            off = (g0 + a) * CHUNK
            pltpu.make_async_copy(sup_s.at[sidx_v.at[a]], rows_a,
                                  sem_ga).wait()
            _scale_chunk(rows_a, w_v, a)
            wr_a = pltpu.async_copy(rows_a, msgs_hbm.at[pl.ds(off, CHUNK)],
                                    sem_wa)
            pltpu.make_async_copy(sup_s.at[sidx_v.at[b]], rows_b,
                                  sem_gb).wait()
            _scale_chunk(rows_b, w_v, b)
            wr_b = pltpu.async_copy(rows_b,
                                    msgs_hbm.at[pl.ds(off + CHUNK, CHUNK)],
                                    sem_wb)
            wr_a.wait()

            @pl.when(t < G // 2 - 1)
            def _():
                pltpu.async_copy(sup_s.at[sidx_v.at[a + 2]], rows_a, sem_ga)

            wr_b.wait()

            @pl.when(t < G // 2 - 1)
            def _():
                pltpu.async_copy(sup_s.at[sidx_v.at[b + 2]], rows_b, sem_gb)

            return carry

        lax.fori_loop(0, G // 2, body, 0)


@functools.partial(
    pl.kernel,
    out_type=jax.ShapeDtypeStruct((NC, N_ROWS_PAD, D), jnp.float32),
    mesh=_mesh,
    scratch_types=[
        pltpu.VMEM_SHARED((N_ROWS_PAD, D), jnp.float32),
        pltpu.VMEM((G, CHUNK), jnp.int32),
        pltpu.VMEM((CHUNK, D), jnp.float32),
        pltpu.VMEM((CHUNK, D), jnp.float32),
        pltpu.SemaphoreType.DMA,
        pltpu.SemaphoreType.DMA,
        pltpu.SemaphoreType.DMA,
        pltpu.SemaphoreType.DMA,
    ],
)
def _sc_scatter(msgs_hbm, dst_hbm, out_hbm,
                acc, didx_v, rows_a, rows_b,
                sem_ra, sem_rb, sem_sa, sem_sb):
    c = lax.axis_index("c")
    s = lax.axis_index("s")
    wid = s * NC + c
    row0 = s * ROWS_PT

    def zero_body(i, carry):
        for j in range(D // L):
            rows_a[i, pl.ds(j * L, L)] = jnp.zeros((L,), jnp.float32)
        return carry

    lax.fori_loop(0, CHUNK, zero_body, 0)
    for t in range(ROWS_PT // CHUNK):
        pltpu.sync_copy(rows_a, acc.at[pl.ds(row0 + t * CHUNK, CHUNK)])
    plsc.subcore_barrier()

    crow0 = wid * N_CHUNKS
    for grp in range(N_GROUPS):
        g0 = crow0 + grp * G
        pltpu.sync_copy(dst_hbm.at[pl.ds(g0, G)], didx_v)
        pltpu.async_copy(msgs_hbm.at[pl.ds(g0 * CHUNK, CHUNK)], rows_a,
                         sem_ra)
        pltpu.async_copy(msgs_hbm.at[pl.ds((g0 + 1) * CHUNK, CHUNK)], rows_b,
                         sem_rb)

        def body(t, carry):
            a = 2 * t
            b = a + 1
            off = (g0 + a) * CHUNK
            pltpu.make_async_copy(msgs_hbm.at[pl.ds(off, CHUNK)], rows_a,
                                  sem_ra).wait()
            sc_a = pltpu.async_copy(rows_a, acc.at[didx_v.at[a]], sem_sa,
                                    add=True)
            pltpu.make_async_copy(msgs_hbm.at[pl.ds(off + CHUNK, CHUNK)],
                                  rows_b, sem_rb).wait()
            sc_a.wait()

            @pl.when(t < G // 2 - 1)
            def _():
                pltpu.async_copy(msgs_hbm.at[pl.ds(off + 2 * CHUNK, CHUNK)],
                                 rows_a, sem_ra)

            sc_b = pltpu.async_copy(rows_b, acc.at[didx_v.at[b]], sem_sb,
                                    add=True)
            sc_b.wait()

            @pl.when(t < G // 2 - 1)
            def _():
                pltpu.async_copy(msgs_hbm.at[pl.ds(off + 3 * CHUNK, CHUNK)],
                                 rows_b, sem_rb)

            return carry

        lax.fori_loop(0, G // 2, body, 0)
    plsc.subcore_barrier()

    pltpu.sync_copy(acc.at[pl.ds(row0, ROWS_PT)],
                    out_hbm.at[c, pl.ds(row0, ROWS_PT)])


_BM = 1000


def _tc_matmul_body(x_ref, w_ref, o_ref):
    o_ref[...] = jnp.dot(x_ref[...], w_ref[...],
                         preferred_element_type=jnp.float32)


def _matmul(x, W):
    return pl.pallas_call(
        _tc_matmul_body,
        grid=(N_NODES // _BM,),
        in_specs=[
            pl.BlockSpec((_BM, D), lambda i: (i, 0)),
            pl.BlockSpec((D, D), lambda i: (0, 0)),
        ],
        out_specs=pl.BlockSpec((_BM, D), lambda i: (i, 0)),
        out_shape=jax.ShapeDtypeStruct((N_NODES, D), jnp.float32),
    )(x, W)


def _tc_combine_relu_matmul_body(p_ref, b_ref, w_ref, o_ref):
    x = p_ref[0] + p_ref[1] + b_ref[...]
    o_ref[...] = jnp.dot(jnp.maximum(x, 0.0), w_ref[...],
                         preferred_element_type=jnp.float32)


def _combine_relu_matmul(p, b, W):
    return pl.pallas_call(
        _tc_combine_relu_matmul_body,
        grid=(N_NODES // _BM,),
        in_specs=[
            pl.BlockSpec((NC, _BM, D), lambda i: (0, i, 0)),
            pl.BlockSpec((1, D), lambda i: (0, 0)),
            pl.BlockSpec((D, D), lambda i: (0, 0)),
        ],
        out_specs=pl.BlockSpec((_BM, D), lambda i: (i, 0)),
        out_shape=jax.ShapeDtypeStruct((N_NODES, D), jnp.float32),
    )(p, b.reshape(1, D), W)


def _tc_combine_body(p_ref, b_ref, o_ref):
    o_ref[...] = p_ref[0] + p_ref[1] + b_ref[...]


def _combine(p, b):
    return pl.pallas_call(
        _tc_combine_body,
        grid=(N_NODES // _BM,),
        in_specs=[
            pl.BlockSpec((NC, _BM, D), lambda i: (0, i, 0)),
            pl.BlockSpec((1, D), lambda i: (0, 0)),
        ],
        out_specs=pl.BlockSpec((_BM, D), lambda i: (i, 0)),
        out_shape=jax.ShapeDtypeStruct((N_NODES, D), jnp.float32),
    )(p, b.reshape(1, D))


def kernel(feat, edge_index, edge_weight, W0, b0, W1, b1):
    src = edge_index[0].astype(jnp.int32)
    dst = edge_index[1].astype(jnp.int32)
    w = edge_weight.astype(jnp.float32)
    pad = E_PAD - N_EDGES
    src = jnp.concatenate([src, jnp.zeros((pad,), jnp.int32)])
    dst = jnp.concatenate([dst, jnp.zeros((pad,), jnp.int32)])
    w = jnp.concatenate([w, jnp.zeros((pad,), jnp.float32)])
    src = src.reshape(E_PAD // CHUNK, CHUNK)
    dst = dst.reshape(E_PAD // CHUNK, CHUNK)
    w = w.reshape(E_PAD // CHUNK, CHUNK)

    sup0 = _matmul(feat, W0)
    m0 = _sc_gather_scale(sup0, src, w)
    p0 = _sc_scatter(m0, dst)
    sup1 = _combine_relu_matmul(p0[:, :N_NODES], b0, W1)
    m1 = _sc_gather_scale(sup1, src, w)
    p1 = _sc_scatter(m1, dst)
    return _combine(p1[:, :N_NODES], b1)
